# Initial kernel scaffold; baseline (speedup 1.0000x reference)
import jax
import jax.numpy as jnp
from jax.experimental import pallas as pl


def kernel(sequence, segment_label, token_table, position_table, segment_table):
    raise NotImplementedError("write your pallas kernel here")


# speedup vs baseline: 5.6780x; 5.6780x over previous
import functools

import jax
import jax.numpy as jnp
from jax import lax
from jax.experimental import pallas as pl
from jax.experimental.pallas import tpu as pltpu
from jax.experimental.pallas import tpu_sc as plsc

B = 1024
L = 512
E = 128
VOCAB = 100000

NC = 2
NS = 16
NW = NC * NS
BPW = B // NW
CL = 128
NLC = L // CL
LANES = 16
EV = E // LANES


def _emb_body(seq_hbm, seg_hbm, tok_hbm, pos_hbm, segtab_hbm, out_hbm,
              idx_v, seg_v, rows_v, pos_v, segtab_v, sem):
    cid = lax.axis_index("c")
    sid = lax.axis_index("s")
    wid = sid * NC + cid

    pltpu.sync_copy(segtab_hbm, segtab_v)
    seg_rows = [[segtab_v[r, pl.ds(j * LANES, LANES)] for j in range(EV)]
                for r in range(3)]
    d1 = [seg_rows[1][j] - seg_rows[0][j] for j in range(EV)]
    d2 = [seg_rows[2][j] - seg_rows[0][j] for j in range(EV)]

    for lc in range(NLC):
        pltpu.sync_copy(pos_hbm.at[pl.ds(lc * CL, CL)], pos_v)

        def pos_body(i, _):
            for j in range(EV):
                sl = pl.ds(j * LANES, LANES)
                pos_v[i, sl] = pos_v[i, sl] + seg_rows[0][j]
            return 0

        lax.fori_loop(0, CL, pos_body, 0)

        def batch_body(bi, _, lc=lc):
            b = wid * BPW + bi
            base = pl.multiple_of(b * L + lc * CL, CL)
            pltpu.sync_copy(seq_hbm.at[pl.ds(base, CL)], idx_v)
            pltpu.sync_copy(seg_hbm.at[pl.ds(base, CL)], seg_v)
            pltpu.async_copy(tok_hbm.at[idx_v], rows_v, sem).wait()

            def group_body(g, _):
                i0 = pl.multiple_of(g * LANES, LANES)
                segf = seg_v[pl.ds(i0, LANES)].astype(jnp.float32)
                m1v = segf * (2.0 - segf)
                m2v = segf * (segf - 1.0) * 0.5
                for k in range(LANES):
                    m1 = jnp.broadcast_to(m1v[k], (LANES,))
                    m2 = jnp.broadcast_to(m2v[k], (LANES,))
                    i = i0 + k
                    for j in range(EV):
                        sl = pl.ds(j * LANES, LANES)
                        rows_v[i, sl] = (rows_v[i, sl] + pos_v[i, sl]
                                         + d1[j] * m1 + d2[j] * m2)
                return 0

            lax.fori_loop(0, CL // LANES, group_body, 0)
            pltpu.sync_copy(rows_v, out_hbm.at[pl.ds(base, CL)])
            return 0

        lax.fori_loop(0, BPW, batch_body, 0)


@functools.partial(jax.jit, static_argnames=())
def kernel(sequence, segment_label, token_table, position_table,
           segment_table):
    seq = sequence.reshape(-1).astype(jnp.int32)
    seg = segment_label.reshape(-1).astype(jnp.int32)

    mesh = plsc.VectorSubcoreMesh(core_axis_name="c", subcore_axis_name="s",
                                  num_cores=NC, num_subcores=NS)
    out = pl.kernel(
        _emb_body,
        out_type=jax.ShapeDtypeStruct((B * L, E), jnp.float32),
        mesh=mesh,
        scratch_types=[
            pltpu.VMEM((CL,), jnp.int32),
            pltpu.VMEM((CL,), jnp.int32),
            pltpu.VMEM((CL, E), jnp.float32),
            pltpu.VMEM((CL, E), jnp.float32),
            pltpu.VMEM((3, E), jnp.float32),
            pltpu.SemaphoreType.DMA,
        ],
    )(seq, seg, token_table, position_table, segment_table)
    return out.reshape(B, L, E)
